# baseline (speedup 1.0000x reference)
import jax
import jax.numpy as jnp
import numpy as np
from jax import lax
from jax.experimental import pallas as pl
from jax.experimental.pallas import tpu as pltpu
from jax.experimental.pallas import tpu_sc as plsc

B, D, L = 16, 64, 4096
HEAD = 64
HPAD = 128
TAIL = L - HPAD
NC, NS = 2, 16
ZPW = D // 2
NBAND = ZPW // 8
LANES = 16
TB = 8

_ZTAIL = np.zeros((8, TAIL), np.float32)


def _sc_body(schemas_hbm, logits_hbm, ztail_hbm, out_hbm,
             logits_v, schemas_v, starts_v, htile_v, zsem, sem):
    cid = lax.axis_index("c")
    sid = lax.axis_index("s")
    wid = sid * NC + cid
    b = wid // 2
    d0 = (wid % 2) * ZPW

    pltpu.sync_copy(schemas_hbm.at[b], schemas_v)
    pltpu.sync_copy(logits_hbm.at[b], logits_v)
    tails = [
        pltpu.async_copy(
            ztail_hbm, out_hbm.at[b, pl.ds(d0 + 8 * t, 8), pl.ds(HPAD, TAIL)], zsem)
        for t in range(NBAND)
    ]
    zv = jnp.zeros((LANES,), jnp.float32)
    for buf in range(2):
        for r in range(8):
            for c4 in range(HEAD // LANES):
                htile_v[buf, r, pl.ds(HEAD + c4 * LANES, LANES)] = zv

    carry = jnp.int32(0)
    for ci in range(D // LANES):
        seg = schemas_v[pl.ds(ci * LANES, LANES)]
        inc = plsc.cumsum(seg)
        starts_v[pl.ds(ci * LANES, LANES)] = inc - seg + carry
        carry = carry + jnp.sum(seg)

    iota = lax.iota(jnp.int32, LANES)
    heads = []
    for t in range(NBAND):
        buf = t % 2
        if t >= 2:
            heads[t - 2].wait()
        for r in range(8):
            idxv = jnp.full((LANES,), d0 + t * 8 + r, jnp.int32)
            nd = plsc.load_gather(schemas_v, [idxv])
            sd = plsc.load_gather(starts_v, [idxv])
            for c4 in range(HEAD // LANES):
                j = iota + (c4 * LANES)
                m = j < nd
                gi = jnp.where(m, sd + j, 0)
                vals = plsc.load_gather(logits_v, [gi])
                htile_v[buf, r, pl.ds(c4 * LANES, LANES)] = jnp.where(m, vals, 0.0)
        heads.append(pltpu.async_copy(
            htile_v.at[buf], out_hbm.at[b, pl.ds(d0 + 8 * t, 8), pl.ds(0, HPAD)],
            sem))
    for cp in heads[-2:]:
        cp.wait()
    for cp in tails:
        cp.wait()


def _tc_mask_body(schemas_ref, outm_ref):
    col = lax.broadcasted_iota(jnp.int32, (B, D, HPAD), 2)
    n = schemas_ref[...]
    outm_ref[...] = (col < n[:, :, None]).astype(jnp.int8)


def kernel(schemas, logits):
    schemas = schemas.astype(jnp.int32)
    logits = logits.astype(jnp.float32)

    mesh = plsc.VectorSubcoreMesh(core_axis_name="c", subcore_axis_name="s",
                                  num_cores=NC, num_subcores=NS)
    sc_out = pl.kernel(
        _sc_body,
        out_type=jax.ShapeDtypeStruct((B, D, L), jnp.float32),
        mesh=mesh,
        compiler_params=pltpu.CompilerParams(use_tc_tiling_on_sc=True,
                                             needs_layout_passes=False),
        scratch_types=[
            pltpu.VMEM((L,), jnp.float32),
            pltpu.VMEM((D,), jnp.int32),
            pltpu.VMEM((D,), jnp.int32),
            pltpu.VMEM((2, 8, HPAD), jnp.float32),
            pltpu.SemaphoreType.DMA,
            pltpu.SemaphoreType.DMA,
        ],
    )
    out_l = sc_out(schemas, logits, jnp.asarray(_ZTAIL))

    mask_head = pl.pallas_call(
        _tc_mask_body,
        out_shape=jax.ShapeDtypeStruct((B, D, HPAD), jnp.int8),
    )(schemas)
    out_m = jnp.concatenate(
        [mask_head.astype(jnp.bool_),
         jnp.zeros((B, D, L - HPAD), jnp.bool_)], axis=-1)
    return out_l, out_m


# speedup vs baseline: 1.8657x; 1.8657x over previous
import jax
import jax.numpy as jnp
import numpy as np
from jax import lax
from jax.experimental import pallas as pl
from jax.experimental.pallas import tpu as pltpu
from jax.experimental.pallas import tpu_sc as plsc

B, D, L = 16, 64, 4096
HEAD = 64
HPAD = 128
TAIL = L - HPAD
NC, NS = 2, 16
LANES = 16
SB = 8
RPW = 16
NBAND = RPW // 8

_ZTAIL = np.zeros((8, TAIL), np.float32)


def _heads_chunk(logits_v, schemas_v, starts_v, zone, iota):
    idxv = jnp.full((LANES,), zone, jnp.int32)
    nd = plsc.load_gather(schemas_v, [idxv])
    sd = plsc.load_gather(starts_v, [idxv])
    out = []
    for c4 in range(HEAD // LANES):
        j = iota + (c4 * LANES)
        m = j < nd
        gi = jnp.where(m, sd + j, 0)
        vals = plsc.load_gather(logits_v, [gi])
        out.append(jnp.where(m, vals, 0.0))
    return out


def _cumsum64(schemas_v, starts_v):
    carry = jnp.int32(0)
    for ci in range(D // LANES):
        seg = schemas_v[pl.ds(ci * LANES, LANES)]
        inc = plsc.cumsum(seg)
        starts_v[pl.ds(ci * LANES, LANES)] = inc - seg + carry
        carry = carry + jnp.sum(seg)


def _sc_body(schemas_hbm, logits_hbm, ztail_hbm, out_hbm, heads_hbm,
             logits_v, schemas_v, starts_v, logits2_v, schemas2_v, starts2_v,
             htile_v, hbuf2_v, zsem, sem):
    cid = lax.axis_index("c")
    sid = lax.axis_index("s")
    wid = sid * NC + cid
    b = wid // 4
    q = wid % 4
    r0 = q * RPW
    bs = b + SB

    pltpu.sync_copy(schemas_hbm.at[b], schemas_v)
    pltpu.sync_copy(logits_hbm.at[b], logits_v)
    tails = [
        pltpu.async_copy(
            ztail_hbm, out_hbm.at[b, pl.ds(r0 + 8 * t, 8), pl.ds(HPAD, TAIL)], zsem)
        for t in range(NBAND)
    ]
    pltpu.sync_copy(schemas_hbm.at[bs], schemas2_v)
    pltpu.sync_copy(logits_hbm.at[bs], logits2_v)

    zv = jnp.zeros((LANES,), jnp.float32)
    for buf in range(2):
        for r in range(8):
            for c4 in range(HEAD // LANES):
                htile_v[buf, r, pl.ds(HEAD + c4 * LANES, LANES)] = zv
    for i in range(RPW):
        for c4 in range(HEAD // LANES):
            hbuf2_v[pl.ds(i * HPAD + HEAD + c4 * LANES, LANES)] = zv

    _cumsum64(schemas_v, starts_v)
    _cumsum64(schemas2_v, starts2_v)

    iota = lax.iota(jnp.int32, LANES)
    heads = []
    for t in range(NBAND):
        buf = t % 2
        for r in range(8):
            qtr = _heads_chunk(logits_v, schemas_v, starts_v, r0 + t * 8 + r, iota)
            for c4, vals in enumerate(qtr):
                htile_v[buf, r, pl.ds(c4 * LANES, LANES)] = vals
        heads.append(pltpu.async_copy(
            htile_v.at[buf], out_hbm.at[b, pl.ds(r0 + 8 * t, 8), pl.ds(0, HPAD)],
            sem))
    for i in range(RPW):
        qtr = _heads_chunk(logits2_v, schemas2_v, starts2_v, r0 + i, iota)
        for c4, vals in enumerate(qtr):
            hbuf2_v[pl.ds(i * HPAD + c4 * LANES, LANES)] = vals
    pltpu.sync_copy(
        hbuf2_v, heads_hbm.at[pl.ds((b * D + r0) * HPAD, RPW * HPAD)])
    for cp in heads:
        cp.wait()
    for cp in tails:
        cp.wait()


def _tc_mask_body(schemas_ref, outm_ref):
    col = lax.broadcasted_iota(jnp.int32, (B, D, HPAD), 2)
    n = schemas_ref[...]
    outm_ref[...] = (col < n[:, :, None]).astype(jnp.int8)


def _tc_assemble_body(aliased_ref, heads_ref, outl_ref):
    del aliased_ref
    outl_ref[0, :, pl.ds(0, HPAD)] = heads_ref[0]
    outl_ref[0, :, pl.ds(HPAD, TAIL)] = jnp.zeros((D, TAIL), jnp.float32)


def kernel(schemas, logits):
    schemas = schemas.astype(jnp.int32)
    logits = logits.astype(jnp.float32)

    mesh = plsc.VectorSubcoreMesh(core_axis_name="c", subcore_axis_name="s",
                                  num_cores=NC, num_subcores=NS)
    sc_out = pl.kernel(
        _sc_body,
        out_type=[jax.ShapeDtypeStruct((B, D, L), jnp.float32),
                  jax.ShapeDtypeStruct(((B - SB) * D * HPAD,), jnp.float32)],
        mesh=mesh,
        compiler_params=pltpu.CompilerParams(use_tc_tiling_on_sc=True,
                                             needs_layout_passes=False),
        scratch_types=[
            pltpu.VMEM((L,), jnp.float32),
            pltpu.VMEM((D,), jnp.int32),
            pltpu.VMEM((D,), jnp.int32),
            pltpu.VMEM((L,), jnp.float32),
            pltpu.VMEM((D,), jnp.int32),
            pltpu.VMEM((D,), jnp.int32),
            pltpu.VMEM((2, 8, HPAD), jnp.float32),
            pltpu.VMEM((RPW * HPAD,), jnp.float32),
            pltpu.SemaphoreType.DMA,
            pltpu.SemaphoreType.DMA,
        ],
    )
    part_l, heads_hi = sc_out(schemas, logits, jnp.asarray(_ZTAIL))
    heads_hi = heads_hi.reshape(B - SB, D, HPAD)

    out_l = pl.pallas_call(
        _tc_assemble_body,
        grid=(B - SB,),
        in_specs=[
            pl.BlockSpec((1, D, HPAD), lambda i: (i + SB, 0, 0)),
            pl.BlockSpec((1, D, HPAD), lambda i: (i, 0, 0)),
        ],
        out_specs=pl.BlockSpec((1, D, L), lambda i: (i + SB, 0, 0)),
        out_shape=jax.ShapeDtypeStruct((B, D, L), jnp.float32),
        input_output_aliases={0: 0},
    )(part_l, heads_hi)

    mask_head = pl.pallas_call(
        _tc_mask_body,
        out_shape=jax.ShapeDtypeStruct((B, D, HPAD), jnp.int8),
    )(schemas)
    out_m = jnp.concatenate(
        [mask_head.astype(jnp.bool_),
         jnp.zeros((B, D, L - HPAD), jnp.bool_)], axis=-1)
    return out_l, out_m
